# baseline (speedup 1.0000x reference)
import functools

import jax
import jax.numpy as jnp
from jax import lax
from jax.experimental import pallas as pl
from jax.experimental.pallas import tpu as pltpu
from jax.experimental.pallas import tpu_sc as plsc

_B = 4096
_M = 4096
_D = 256
_K = 16
_TAU = 0.3

_NC = 2
_NS = 16
_NW = _NC * _NS
_ROWS = _B // _NW
_GROUPS = _M // 32
_NSTR = 4

_INF = float("inf")


def _merge16(bk, bv, ck, cv):
    sk, sv = plsc.sort_key_val(ck, cv)
    rk = lax.rev(sk, (0,))
    rv = lax.rev(sv, (0,))
    keep = bk <= rk
    nk = jnp.minimum(bk, rk)
    nv = jnp.where(keep, bv, rv)
    return plsc.sort_key_val(nk, nv)


_NDB = _NSTR - 1


def _sc_topk(q_c, anc_c):
    mesh = plsc.VectorSubcoreMesh(
        core_axis_name="c", subcore_axis_name="s",
        num_cores=_NC, num_subcores=_NS)

    f32 = jnp.float32
    out_sds = jax.ShapeDtypeStruct((_B, _K), f32)
    nbufs = 2 * _NDB + 1

    @functools.partial(
        pl.kernel, mesh=mesh,
        out_type=(out_sds,) * 3,
        compiler_params=pltpu.CompilerParams(needs_layout_passes=False),
        scratch_types=(
            [pltpu.VMEM((2, _M), f32)] * nbufs
            + [pltpu.VMEM((_ROWS, _K), f32)] * 3
            + [pltpu.SemaphoreType.DMA] * nbufs
        ),
    )
    def k(q_h, anc_h, ox_h, oy_h, ow_h, *scr):
        allbufs = scr[:nbufs]
        oxb, oyb, owb = scr[nbufs:nbufs + 3]
        sems = scr[nbufs + 3:]
        wid = lax.axis_index("s") * _NC + lax.axis_index("c")
        base = wid * _ROWS
        seg = _ROWS // _NSTR

        iota = lax.broadcasted_iota(jnp.int32, (_K,), 0)
        zero16 = jnp.zeros((_K,), jnp.int32)
        one16 = jnp.full((_K,), 1, jnp.int32)
        streams = [
            ((allbufs[2 * s], sems[2 * s]), (allbufs[2 * s + 1], sems[2 * s + 1]))
            for s in range(_NDB)
        ]
        lastbuf, lastsem = allbufs[-1], sems[-1]

        if True:
            pltpu.sync_copy(q_h.at[pl.ds(base, _ROWS)], oxb)
            for s in range(_NDB):
                buf, sem = streams[s][0]
                pltpu.async_copy(anc_h.at[base + s * seg], buf, sem)
            pltpu.async_copy(anc_h.at[base + _NDB * seg], lastbuf, lastsem)

            def compute_rows(rr, bufs):
                rows = [rr + s * seg for s in range(_NSTR)]
                qvs = [oxb[r] for r in rows]
                qs = [(jnp.broadcast_to(v[0], (_K,)),
                       jnp.broadcast_to(v[1], (_K,))) for v in qvs]

                def dists(buf, qx, qy, o):
                    dx = buf[0, pl.ds(o, _K)] - qx
                    dy = buf[1, pl.ds(o, _K)] - qy
                    return dx * dx + dy * dy

                def screen_merge(cc, o, bk, bv, wth):
                    m0 = jnp.minimum(jnp.minimum(cc[0], cc[1]),
                                     jnp.minimum(cc[2], cc[3]))
                    m1 = jnp.minimum(jnp.minimum(cc[4], cc[5]),
                                     jnp.minimum(cc[6], cc[7]))
                    mn = jnp.minimum(m0, m1)
                    hits = plsc.all_reduce_population_count(mn < wth)[0]

                    def do_merge(args):
                        bk, bv = args
                        for j in range(8):
                            bk, bv = _merge16(bk, bv, cc[j],
                                              o + j * _K + iota)
                        return bk, bv, bk[_K - 1]

                    def no_merge(args):
                        bk, bv = args
                        return bk, bv, wth

                    return lax.cond(hits > 0, do_merge, no_merge, (bk, bv))

                def group_body(g, carry):
                    o = g * 128
                    cs = [
                        [dists(bufs[s], qs[s][0], qs[s][1], o + j * _K)
                         for j in range(8)]
                        for s in range(_NSTR)
                    ]
                    out = []
                    for s in range(_NSTR):
                        bk, bv, wth = carry[3 * s:3 * s + 3]
                        out.extend(screen_merge(cs[s], o, bk, bv, wth))
                    return tuple(out)

                bk0 = jnp.full((_K,), _INF, f32)
                bv0 = jnp.zeros((_K,), jnp.int32)
                inf = jnp.float32(_INF)
                fin = lax.fori_loop(
                    0, _M // 128, group_body,
                    (bk0, bv0, inf) * _NSTR, unroll=1)

                for s in range(_NSTR):
                    bk, bv, wm = fin[3 * s:3 * s + 3]
                    e = jnp.exp((bk - wm) * (1.0 / _TAU))
                    oxb[rows[s]] = plsc.load_gather(bufs[s], [zero16, bv])
                    oyb[rows[s]] = plsc.load_gather(bufs[s], [one16, bv])
                    owb[rows[s]] = e

            def row_pair(rr2, _, anc_h=anc_h):
                for par in range(2):
                    rr = 2 * rr2 + par

                    @pl.when(rr + 1 < seg)
                    def _():
                        for s in range(_NDB):
                            nbuf, nsem = streams[s][1 - par]
                            pltpu.async_copy(
                                anc_h.at[base + s * seg + rr + 1], nbuf, nsem)

                    bufs = []
                    for s in range(_NDB):
                        buf, sem = streams[s][par]
                        pltpu.make_async_copy(anc_h.at[base + s * seg + rr],
                                              buf, sem).wait()
                        bufs.append(buf)
                    pltpu.make_async_copy(anc_h.at[base + _NDB * seg + rr],
                                          lastbuf, lastsem).wait()
                    bufs.append(lastbuf)
                    compute_rows(rr, bufs)

                    @pl.when(rr + 1 < seg)
                    def _():
                        pltpu.async_copy(
                            anc_h.at[base + _NDB * seg + rr + 1],
                            lastbuf, lastsem)
                return 0

            lax.fori_loop(0, seg // 2, row_pair, 0)
            pltpu.sync_copy(oxb, ox_h.at[pl.ds(base, _ROWS)])
            pltpu.sync_copy(oyb, oy_h.at[pl.ds(base, _ROWS)])
            pltpu.sync_copy(owb, ow_h.at[pl.ds(base, _ROWS)])

    return k(q_c, anc_c)


def _gelu(x):
    return 0.5 * x * (1.0 + lax.erf(x * (1.0 / jnp.sqrt(2.0).astype(x.dtype))))


_CH = 8192


def _mlp_body(x_ref, w1_ref, b1_ref, w2_ref, b2_ref, o_ref, esum_ref):
    i = pl.program_id(0)
    ni = pl.num_programs(0)
    x = x_ref[...]
    a = x[:, 0:2]
    wgt = x[:, 2:3]
    h1 = _gelu(jnp.dot(a, w1_ref[...], preferred_element_type=jnp.float32)
               + b1_ref[...])
    h2 = _gelu(jnp.dot(h1, w2_ref[...], preferred_element_type=jnp.float32)
               + b2_ref[...])
    h2 = h2 * wgt

    @pl.when(i == 0)
    def _():
        o_ref[...] = jnp.zeros_like(o_ref)
        esum_ref[...] = jnp.zeros_like(esum_ref)

    o_ref[...] += h2[0:_B, :] + h2[_B:_CH, :]
    esum_ref[...] += wgt[0:_B, :] + wgt[_B:_CH, :]

    @pl.when(i == ni - 1)
    def _():
        o_ref[...] = o_ref[...] / esum_ref[...]


def _tc_mlp(x, w1t, b1, w2t, b2):
    grid = (_K * _B) // _CH
    return pl.pallas_call(
        _mlp_body,
        grid=(grid,),
        in_specs=[
            pl.BlockSpec((_CH, 4), lambda i: (i, 0)),
            pl.BlockSpec((2, _D), lambda i: (0, 0)),
            pl.BlockSpec((1, _D), lambda i: (0, 0)),
            pl.BlockSpec((_D, _D), lambda i: (0, 0)),
            pl.BlockSpec((1, _D), lambda i: (0, 0)),
        ],
        out_specs=pl.BlockSpec((_B, _D), lambda i: (0, 0)),
        out_shape=jax.ShapeDtypeStruct((_B, _D), jnp.float32),
        scratch_shapes=[pltpu.VMEM((_B, 1), jnp.float32)],
    )(x, w1t, b1, w2t, b2)


def kernel(nodes_2x2, ancS, ancL, W1, b1, W2, b2):
    gs = nodes_2x2[:, 0, :]
    gl = nodes_2x2[:, 1, :]
    anc_s = ancS.swapaxes(1, 2)
    anc_l = ancL.swapaxes(1, 2)
    pad = jnp.zeros((_B, _K - 2), jnp.float32)
    qp_s = jnp.concatenate([gs, pad], axis=1)
    qp_l = jnp.concatenate([gl, pad], axis=1)

    oxs, oys, ows = _sc_topk(qp_s, anc_s)
    oxl, oyl, owl = _sc_topk(qp_l, anc_l)

    w1t = W1.T
    w2t = W2.T
    b1r = b1.reshape(1, _D)
    b2r = b2.reshape(1, _D)

    def assemble(ox, oy, ow):
        cols = [ox.T.reshape(-1), oy.T.reshape(-1), ow.T.reshape(-1),
                jnp.zeros((_K * _B,), jnp.float32)]
        return jnp.stack(cols, axis=-1)

    hs = _tc_mlp(assemble(oxs, oys, ows), w1t, b1r, w2t, b2r)
    hl = _tc_mlp(assemble(oxl, oyl, owl), w1t, b1r, w2t, b2r)
    return (hs, hl)


# speedup vs baseline: 1.9094x; 1.9094x over previous
import functools

import jax
import jax.numpy as jnp
from jax import lax
from jax.experimental import pallas as pl
from jax.experimental.pallas import tpu as pltpu
from jax.experimental.pallas import tpu_sc as plsc

_B = 4096
_M = 4096
_D = 256
_K = 16
_TAU = 0.3

_NC = 2
_NS = 16
_NW = _NC * _NS
_ROWS = _B // _NW
_GROUPS = _M // 32
_NSTR = 4

_INF = float("inf")


def _merge16(bk, bv, ck, cv):
    sk, sv = plsc.sort_key_val(ck, cv)
    rk = lax.rev(sk, (0,))
    rv = lax.rev(sv, (0,))
    keep = bk <= rk
    nk = jnp.minimum(bk, rk)
    nv = jnp.where(keep, bv, rv)
    return plsc.sort_key_val(nk, nv)


_NDB = _NSTR - 1


def _sc_topk(q_c, anc_c):
    mesh = plsc.VectorSubcoreMesh(
        core_axis_name="c", subcore_axis_name="s",
        num_cores=_NC, num_subcores=_NS)

    f32 = jnp.float32
    out_sds = jax.ShapeDtypeStruct((_B, _K), f32)
    nbufs = 2 * _NDB + 1

    @functools.partial(
        pl.kernel, mesh=mesh,
        out_type=(out_sds,) * 3,
        compiler_params=pltpu.CompilerParams(needs_layout_passes=False),
        scratch_types=(
            [pltpu.VMEM((2, _M), f32)] * nbufs
            + [pltpu.VMEM((_ROWS, _K), f32)] * 3
            + [pltpu.SemaphoreType.DMA] * nbufs
        ),
    )
    def k(q_h, anc_h, ox_h, oy_h, ow_h, *scr):
        allbufs = scr[:nbufs]
        oxb, oyb, owb = scr[nbufs:nbufs + 3]
        sems = scr[nbufs + 3:]
        wid = lax.axis_index("s") * _NC + lax.axis_index("c")
        base = wid * _ROWS
        seg = _ROWS // _NSTR

        iota = lax.broadcasted_iota(jnp.int32, (_K,), 0)
        zero16 = jnp.zeros((_K,), jnp.int32)
        one16 = jnp.full((_K,), 1, jnp.int32)
        streams = [
            ((allbufs[2 * s], sems[2 * s]), (allbufs[2 * s + 1], sems[2 * s + 1]))
            for s in range(_NDB)
        ]
        lastbuf, lastsem = allbufs[-1], sems[-1]

        if True:
            pltpu.sync_copy(q_h.at[pl.ds(base, _ROWS)], oxb)
            for s in range(_NDB):
                buf, sem = streams[s][0]
                pltpu.async_copy(anc_h.at[base + s * seg], buf, sem)
            pltpu.async_copy(anc_h.at[base + _NDB * seg], lastbuf, lastsem)

            def compute_rows(rr, bufs):
                rows = [rr + s * seg for s in range(_NSTR)]
                qvs = [oxb[r] for r in rows]
                qs = [(jnp.broadcast_to(v[0], (_K,)),
                       jnp.broadcast_to(v[1], (_K,))) for v in qvs]

                def dists(buf, qx, qy, o):
                    dx = buf[0, pl.ds(o, _K)] - qx
                    dy = buf[1, pl.ds(o, _K)] - qy
                    return dx * dx + dy * dy

                def screen_merge(cc, o, bk, bv, wth):
                    mn = jnp.minimum(jnp.minimum(cc[0], cc[1]),
                                     jnp.minimum(cc[2], cc[3]))
                    hits = plsc.all_reduce_population_count(mn < wth)[0]

                    def do_merge(args):
                        bk, bv = args
                        for j in range(4):
                            bk, bv = _merge16(bk, bv, cc[j],
                                              o + j * _K + iota)
                        return bk, bv, bk[_K - 1]

                    def no_merge(args):
                        bk, bv = args
                        return bk, bv, wth

                    return lax.cond(hits > 0, do_merge, no_merge, (bk, bv))

                def group_body(g, carry):
                    o = g * 64
                    cs = [
                        [dists(bufs[s], qs[s][0], qs[s][1], o + j * _K)
                         for j in range(4)]
                        for s in range(_NSTR)
                    ]
                    out = []
                    for s in range(_NSTR):
                        bk, bv, wth = carry[3 * s:3 * s + 3]
                        out.extend(screen_merge(cs[s], o, bk, bv, wth))
                    return tuple(out)

                bk0 = jnp.full((_K,), _INF, f32)
                bv0 = jnp.zeros((_K,), jnp.int32)
                inf = jnp.float32(_INF)
                fin = lax.fori_loop(
                    0, _M // 64, group_body,
                    (bk0, bv0, inf) * _NSTR, unroll=1)

                for s in range(_NSTR):
                    bk, bv, wm = fin[3 * s:3 * s + 3]
                    e = jnp.exp((bk - wm) * (1.0 / _TAU))
                    oxb[rows[s]] = plsc.load_gather(bufs[s], [zero16, bv])
                    oyb[rows[s]] = plsc.load_gather(bufs[s], [one16, bv])
                    owb[rows[s]] = e

            def row_pair(rr2, _, anc_h=anc_h):
                for par in range(2):
                    rr = 2 * rr2 + par

                    @pl.when(rr + 1 < seg)
                    def _():
                        for s in range(_NDB):
                            nbuf, nsem = streams[s][1 - par]
                            pltpu.async_copy(
                                anc_h.at[base + s * seg + rr + 1], nbuf, nsem)

                    bufs = []
                    for s in range(_NDB):
                        buf, sem = streams[s][par]
                        pltpu.make_async_copy(anc_h.at[base + s * seg + rr],
                                              buf, sem).wait()
                        bufs.append(buf)
                    pltpu.make_async_copy(anc_h.at[base + _NDB * seg + rr],
                                          lastbuf, lastsem).wait()
                    bufs.append(lastbuf)
                    compute_rows(rr, bufs)

                    @pl.when(rr + 1 < seg)
                    def _():
                        pltpu.async_copy(
                            anc_h.at[base + _NDB * seg + rr + 1],
                            lastbuf, lastsem)
                return 0

            lax.fori_loop(0, seg // 2, row_pair, 0)
            pltpu.sync_copy(oxb, ox_h.at[pl.ds(base, _ROWS)])
            pltpu.sync_copy(oyb, oy_h.at[pl.ds(base, _ROWS)])
            pltpu.sync_copy(owb, ow_h.at[pl.ds(base, _ROWS)])

    return k(q_c, anc_c)


def _gelu(x):
    return 0.5 * x * (1.0 + lax.erf(x * (1.0 / jnp.sqrt(2.0).astype(x.dtype))))


_CH = 8192


def _mlp_body(x_ref, w1_ref, b1_ref, w2_ref, b2_ref, o_ref, esum_ref):
    i = pl.program_id(0)
    ni = pl.num_programs(0)
    x = x_ref[...]
    a = x[:, 0:2]
    wgt = x[:, 2:3]
    h1 = _gelu(jnp.dot(a, w1_ref[...], preferred_element_type=jnp.float32)
               + b1_ref[...])
    h2 = _gelu(jnp.dot(h1, w2_ref[...], preferred_element_type=jnp.float32)
               + b2_ref[...])
    h2 = h2 * wgt

    @pl.when(i == 0)
    def _():
        o_ref[...] = jnp.zeros_like(o_ref)
        esum_ref[...] = jnp.zeros_like(esum_ref)

    o_ref[...] += h2[0:_B, :] + h2[_B:_CH, :]
    esum_ref[...] += wgt[0:_B, :] + wgt[_B:_CH, :]

    @pl.when(i == ni - 1)
    def _():
        o_ref[...] = o_ref[...] / esum_ref[...]


def _tc_mlp(x, w1t, b1, w2t, b2):
    grid = (_K * _B) // _CH
    return pl.pallas_call(
        _mlp_body,
        grid=(grid,),
        in_specs=[
            pl.BlockSpec((_CH, 4), lambda i: (i, 0)),
            pl.BlockSpec((2, _D), lambda i: (0, 0)),
            pl.BlockSpec((1, _D), lambda i: (0, 0)),
            pl.BlockSpec((_D, _D), lambda i: (0, 0)),
            pl.BlockSpec((1, _D), lambda i: (0, 0)),
        ],
        out_specs=pl.BlockSpec((_B, _D), lambda i: (0, 0)),
        out_shape=jax.ShapeDtypeStruct((_B, _D), jnp.float32),
        scratch_shapes=[pltpu.VMEM((_B, 1), jnp.float32)],
    )(x, w1t, b1, w2t, b2)


def kernel(nodes_2x2, ancS, ancL, W1, b1, W2, b2):
    gs = nodes_2x2[:, 0, :]
    gl = nodes_2x2[:, 1, :]
    anc_s = ancS.swapaxes(1, 2)
    anc_l = ancL.swapaxes(1, 2)
    pad = jnp.zeros((_B, _K - 2), jnp.float32)
    qp_s = jnp.concatenate([gs, pad], axis=1)
    qp_l = jnp.concatenate([gl, pad], axis=1)

    oxs, oys, ows = _sc_topk(qp_s, anc_s)
    oxl, oyl, owl = _sc_topk(qp_l, anc_l)

    w1t = W1.T
    w2t = W2.T
    b1r = b1.reshape(1, _D)
    b2r = b2.reshape(1, _D)

    def assemble(ox, oy, ow):
        cols = [ox.T.reshape(-1), oy.T.reshape(-1), ow.T.reshape(-1),
                jnp.zeros((_K * _B,), jnp.float32)]
        return jnp.stack(cols, axis=-1)

    hs = _tc_mlp(assemble(oxs, oys, ows), w1t, b1r, w2t, b2r)
    hl = _tc_mlp(assemble(oxl, oyl, owl), w1t, b1r, w2t, b2r)
    return (hs, hl)
